# baseline (speedup 1.0000x reference)
import functools

import jax
import jax.numpy as jnp
from jax import lax
from jax.experimental import pallas as pl
from jax.experimental.pallas import tpu as pltpu
from jax.experimental.pallas import tpu_sc as plsc

N = 2097152
R, C = 16384, 128
NC, NS = 2, 16
NW = NC * NS
RPW = R // NW
CR = 128
NCHUNK = RPW // CR

_mesh = plsc.VectorSubcoreMesh(
    core_axis_name="c", subcore_axis_name="s", num_cores=NC, num_subcores=NS)


@functools.partial(
    pl.kernel,
    out_type=jax.ShapeDtypeStruct((R, C), jnp.float32),
    mesh=_mesh,
    scratch_types=[
        pltpu.VMEM((CR, C), jnp.float32),
        pltpu.VMEM((CR, C), jnp.float32),
        pltpu.VMEM((CR, C), jnp.float32),
        pltpu.VMEM((CR, C), jnp.float32),
        pltpu.SemaphoreType.DMA,
        pltpu.SemaphoreType.DMA,
        pltpu.SemaphoreType.DMA,
        pltpu.SemaphoreType.DMA,
    ],
    compiler_params=pltpu.CompilerParams(needs_layout_passes=False),
)
def _vq_vals(x_hbm, vals_hbm, x_v0, x_v1, o_v0, o_v1,
             si0, si1, so0, so1):
    wid = lax.axis_index("s") * NC + lax.axis_index("c")
    rbase = wid * RPW
    x_bufs, o_bufs = (x_v0, x_v1), (o_v0, o_v1)
    si, so = (si0, si1), (so0, so1)

    def row_slice(ci):
        return pl.ds(pl.multiple_of(rbase + ci * CR, CR), CR)

    def start_in(ci):
        return pltpu.async_copy(
            x_hbm.at[row_slice(ci), :], x_bufs[ci % 2], si[ci % 2])

    in_copies = {0: start_in(0)}
    out_copies = {}
    for ci in range(NCHUNK):
        if ci + 1 < NCHUNK:
            in_copies[ci + 1] = start_in(ci + 1)
        in_copies[ci].wait()
        if ci >= 2:
            out_copies[ci - 2].wait()
        x_v, vals_v = x_bufs[ci % 2], o_bufs[ci % 2]

        @plsc.parallel_loop(0, CR)
        def row_body(r):
            for c in range(C // 16):
                xv = x_v[r, pl.ds(c * 16, 16)]
                u = jnp.clip(8.0 - xv, 0.0, 15.0).astype(jnp.int32)
                vals_v[r, pl.ds(c * 16, 16)] = 7.5 - u.astype(jnp.float32)

        out_copies[ci] = pltpu.async_copy(
            vals_v, vals_hbm.at[row_slice(ci), :], so[ci % 2])
    out_copies[NCHUNK - 2].wait()
    out_copies[NCHUNK - 1].wait()


_BR = 1024
_G = R // _BR


def _idx_body(x_hbm, idx_ref, x_v, sems):
    i = pl.program_id(0)

    @pl.when(i == 0)
    def _prime():
        pltpu.make_async_copy(
            x_hbm.at[pl.ds(0, _BR), :], x_v.at[0], sems.at[0]).start()

    @pl.when(i + 1 < _G)
    def _next():
        pltpu.make_async_copy(
            x_hbm.at[pl.ds((i + 1) * _BR, _BR), :],
            x_v.at[(i + 1) % 2], sems.at[(i + 1) % 2]).start()

    pltpu.make_async_copy(
        x_hbm.at[pl.ds(i * _BR, _BR), :], x_v.at[i % 2], sems.at[i % 2]).wait()
    u = jnp.clip(8.0 - x_v[i % 2], 0.0, 15.0).astype(jnp.int32)
    idx_ref[...] = (15 - u).astype(jnp.uint8)


_vq_idx = pl.pallas_call(
    _idx_body,
    grid=(_G,),
    in_specs=[pl.BlockSpec(memory_space=pl.ANY)],
    out_specs=pl.BlockSpec((_BR, C), lambda i: (i, 0)),
    out_shape=jax.ShapeDtypeStruct((R, C), jnp.uint8),
    scratch_shapes=[
        pltpu.VMEM((2, _BR, C), jnp.float32),
        pltpu.SemaphoreType.DMA((2,)),
    ],
    compiler_params=pltpu.CompilerParams(
        dimension_semantics=("arbitrary",)),
)


def kernel(X, grid, grid_norm):
    x2d = X.reshape(R, C)
    vals2d = _vq_vals(x2d)
    idx2d = _vq_idx(pltpu.with_memory_space_constraint(x2d, pltpu.MemorySpace.HBM))
    return (vals2d.reshape(N, 1), idx2d.reshape(N))


# speedup vs baseline: 1.0232x; 1.0232x over previous
import functools

import jax
import jax.numpy as jnp
from jax import lax
from jax.experimental import pallas as pl
from jax.experimental.pallas import tpu as pltpu
from jax.experimental.pallas import tpu_sc as plsc

N = 2097152
R, C = 16384, 128
NC, NS = 2, 16
NW = NC * NS
RPW = R // NW
CR = 128
NCHUNK = RPW // CR

_mesh = plsc.VectorSubcoreMesh(
    core_axis_name="c", subcore_axis_name="s", num_cores=NC, num_subcores=NS)


@functools.partial(
    pl.kernel,
    out_type=jax.ShapeDtypeStruct((R, C), jnp.float32),
    mesh=_mesh,
    scratch_types=[
        pltpu.VMEM((CR, C), jnp.float32),
        pltpu.VMEM((CR, C), jnp.float32),
        pltpu.VMEM((CR, C), jnp.float32),
        pltpu.VMEM((CR, C), jnp.float32),
        pltpu.SemaphoreType.DMA,
        pltpu.SemaphoreType.DMA,
        pltpu.SemaphoreType.DMA,
        pltpu.SemaphoreType.DMA,
    ],
    compiler_params=pltpu.CompilerParams(needs_layout_passes=False),
)
def _vq_vals(x_hbm, vals_hbm, x_v0, x_v1, o_v0, o_v1,
             si0, si1, so0, so1):
    wid = lax.axis_index("s") * NC + lax.axis_index("c")
    rbase = wid * RPW
    x_bufs, o_bufs = (x_v0, x_v1), (o_v0, o_v1)
    si, so = (si0, si1), (so0, so1)

    def row_slice(ci):
        return pl.ds(pl.multiple_of(rbase + ci * CR, CR), CR)

    def start_in(ci):
        return pltpu.async_copy(
            x_hbm.at[row_slice(ci), :], x_bufs[ci % 2], si[ci % 2])

    in_copies = {0: start_in(0)}
    out_copies = {}
    for ci in range(NCHUNK):
        if ci + 1 < NCHUNK:
            in_copies[ci + 1] = start_in(ci + 1)
        in_copies[ci].wait()
        if ci >= 2:
            out_copies[ci - 2].wait()
        x_v, vals_v = x_bufs[ci % 2], o_bufs[ci % 2]

        @plsc.parallel_loop(0, CR)
        def row_body(r):
            for c in range(C // 16):
                xv = x_v[r, pl.ds(c * 16, 16)]
                u = jnp.clip(8.0 - xv, 0.0, 15.0).astype(jnp.int32)
                vals_v[r, pl.ds(c * 16, 16)] = 7.5 - u.astype(jnp.float32)

        out_copies[ci] = pltpu.async_copy(
            vals_v, vals_hbm.at[row_slice(ci), :], so[ci % 2])
    out_copies[NCHUNK - 2].wait()
    out_copies[NCHUNK - 1].wait()


_BR = 2048
_G = R // _BR


def _idx_body(x_hbm, idx_ref, x_v, sems):
    i = pl.program_id(0)

    @pl.when(i == 0)
    def _prime():
        pltpu.make_async_copy(
            x_hbm.at[pl.ds(0, _BR), :], x_v.at[0], sems.at[0]).start()

    @pl.when(i + 1 < _G)
    def _next():
        pltpu.make_async_copy(
            x_hbm.at[pl.ds((i + 1) * _BR, _BR), :],
            x_v.at[(i + 1) % 2], sems.at[(i + 1) % 2]).start()

    pltpu.make_async_copy(
        x_hbm.at[pl.ds(i * _BR, _BR), :], x_v.at[i % 2], sems.at[i % 2]).wait()
    u = jnp.clip(8.0 - x_v[i % 2], 0.0, 15.0).astype(jnp.int32)
    idx_ref[...] = (15 - u).astype(jnp.uint8)


_vq_idx = pl.pallas_call(
    _idx_body,
    grid=(_G,),
    in_specs=[pl.BlockSpec(memory_space=pl.ANY)],
    out_specs=pl.BlockSpec((_BR, C), lambda i: (i, 0)),
    out_shape=jax.ShapeDtypeStruct((R, C), jnp.uint8),
    scratch_shapes=[
        pltpu.VMEM((2, _BR, C), jnp.float32),
        pltpu.SemaphoreType.DMA((2,)),
    ],
    compiler_params=pltpu.CompilerParams(
        dimension_semantics=("arbitrary",)),
)


def kernel(X, grid, grid_norm):
    x2d = X.reshape(R, C)
    vals2d = _vq_vals(x2d)
    idx2d = _vq_idx(pltpu.with_memory_space_constraint(x2d, pltpu.MemorySpace.HBM))
    return (vals2d.reshape(N, 1), idx2d.reshape(N))
